# baseline (speedup 1.0000x reference)
import functools

import jax
import jax.numpy as jnp
from jax import lax
from jax.experimental import pallas as pl
from jax.experimental.pallas import tpu as pltpu
from jax.experimental.pallas import tpu_sc as plsc

B = 8192
D = 256
K = 10
N = 50000
NC = 2
NS = 16
NW = NC * NS
BPW = B // NW
C = 8
NCH = BPW // C
SC = 64
NSC = BPW // SC


def _reduce_chunk(nbuf, sumbuf):
    def node_body(r, carry):
        rb = r * K
        for d in range(D // 16):
            acc = nbuf[rb, pl.ds(d * 16, 16)]
            for j in range(1, K):
                acc = acc + nbuf[rb + j, pl.ds(d * 16, 16)]
            sumbuf[r, pl.ds(d * 16, 16)] = acc
        return carry
    lax.fori_loop(0, C, node_body, 0)


RING = 4


def _sc_gather_body(nodes_h, nmap_h, nidx_h, feat_h, self_o, sum_o,
                    nid_v, map_v, nbr_v, sbuf0, sbuf1,
                    nbuf0, nbuf1, nbuf2, nbuf3,
                    qbuf0, qbuf1, semi, sem_s0, sem_s1,
                    sem_n0, sem_n1, sem_n2, sem_n3, sem_o0, sem_o1):
    wid = lax.axis_index("s") * NC + lax.axis_index("c")
    base = wid * BPW

    cp_nbr = pltpu.async_copy(nidx_h.at[wid], nbr_v, semi)
    pltpu.sync_copy(nodes_h.at[pl.ds(base, BPW)], nid_v)
    pltpu.async_copy(nmap_h.at[nid_v], map_v, sem_s0).wait()
    cp_nbr.wait()

    nbufs = (nbuf0, nbuf1, nbuf2, nbuf3)
    nsems = (sem_n0, sem_n1, sem_n2, sem_n3)

    for b in range(RING):
        pltpu.async_copy(feat_h.at[nbr_v.at[b]], nbufs[b], nsems[b])

    sbufs, ssems = (sbuf0, sbuf1), (sem_s0, sem_s1)
    gathers = []
    for s in range(2):
        gathers.append(pltpu.async_copy(
            feat_h.at[map_v.at[pl.ds(s * SC, SC)]], sbufs[s], ssems[s]))
    outs = [None, None]
    for s in range(NSC):
        b = s % 2
        gathers[s].wait()
        if outs[b] is not None:
            outs[b].wait()
        outs[b] = pltpu.async_copy(
            sbufs[b], self_o.at[pl.ds(base + s * SC, SC)], ssems[b])
        if s + 2 < NSC:
            if outs[b] is not None:
                outs[b].wait()
                outs[b] = None
            gathers.append(pltpu.async_copy(
                feat_h.at[map_v.at[pl.ds((s + 2) * SC, SC)]],
                sbufs[b], ssems[b]))
    for o in outs:
        o.wait()

    qbufs = (qbuf0, qbuf1)
    osems = (sem_o0, sem_o1)

    def ring_body(p, carry):
        cc = p * RING
        for b in range(RING):
            c = cc + b
            q = b % 2
            pltpu.make_async_copy(feat_h.at[nbr_v.at[c]], nbufs[b],
                                  nsems[b]).wait()
            @pl.when(c >= 2)
            def _wait_prev():
                pltpu.make_async_copy(
                    qbufs[q], sum_o.at[pl.ds(base + (c - 2) * C, C)],
                    osems[q]).wait()
            _reduce_chunk(nbufs[b], qbufs[q])

            @pl.when(c + RING < NCH)
            def _next_gather():
                pltpu.async_copy(feat_h.at[nbr_v.at[c + RING]], nbufs[b],
                                 nsems[b])
            pltpu.async_copy(qbufs[q], sum_o.at[pl.ds(base + c * C, C)],
                             osems[q])
        return carry

    lax.fori_loop(0, NCH // RING, ring_body, 0)
    for b in range(2):
        pltpu.make_async_copy(
            qbufs[b], sum_o.at[pl.ds(base + (NCH - 2 + b) * C, C)],
            osems[b]).wait()


@functools.cache
def _make_sc_gather():
    return pl.kernel(
        _sc_gather_body,
        mesh=plsc.VectorSubcoreMesh(core_axis_name="c", subcore_axis_name="s"),
        out_type=[
            jax.ShapeDtypeStruct((B, D), jnp.float32),
            jax.ShapeDtypeStruct((B, D), jnp.float32),
        ],
        scratch_types=[
            pltpu.VMEM((BPW,), jnp.int32),
            pltpu.VMEM((BPW,), jnp.int32),
            pltpu.VMEM((NCH, C * K), jnp.int32),
            pltpu.VMEM((SC, D), jnp.float32),
            pltpu.VMEM((SC, D), jnp.float32),
            pltpu.VMEM((C * K, D), jnp.float32),
            pltpu.VMEM((C * K, D), jnp.float32),
            pltpu.VMEM((C * K, D), jnp.float32),
            pltpu.VMEM((C * K, D), jnp.float32),
            pltpu.VMEM((C, D), jnp.float32),
            pltpu.VMEM((C, D), jnp.float32),
            pltpu.SemaphoreType.DMA,
            pltpu.SemaphoreType.DMA,
            pltpu.SemaphoreType.DMA,
            pltpu.SemaphoreType.DMA,
            pltpu.SemaphoreType.DMA,
            pltpu.SemaphoreType.DMA,
            pltpu.SemaphoreType.DMA,
            pltpu.SemaphoreType.DMA,
            pltpu.SemaphoreType.DMA,
        ],
    )


NB = 8
BB = B // NB


def _tc_body(self_ref, sum_ref, w_ref, g_ref, b_ref, out_ref,
             ht_ref, s1_ref, s2_ref):
    i = pl.program_id(0)

    @pl.when(i < NB)
    def _matmul():
        ws = w_ref[:, :D]
        wn = w_ref[:, D:]
        dn = (((1,), (1,)), ((), ()))
        blk = lax.dot_general(ws, self_ref[...], dn,
                              preferred_element_type=jnp.float32)
        blk = blk + 0.1 * lax.dot_general(wn, sum_ref[...], dn,
                                          preferred_element_type=jnp.float32)
        ht_ref[:, pl.ds(i * BB, BB)] = blk
        ps = jnp.sum(blk, axis=1, keepdims=True)
        pq = jnp.sum(blk * blk, axis=1, keepdims=True)

        @pl.when(i == 0)
        def _init():
            s1_ref[...] = ps
            s2_ref[...] = pq

        @pl.when(i > 0)
        def _acc():
            s1_ref[...] = s1_ref[...] + ps
            s2_ref[...] = s2_ref[...] + pq

    @pl.when(i >= NB)
    def _bn():
        j = i - NB
        mean = s1_ref[...] * (1.0 / B)
        var = s2_ref[...] * (1.0 / B) - mean * mean
        inv = lax.rsqrt(var + 1e-5)
        blk = ht_ref[:, pl.ds(j * BB, BB)]
        out_ref[...] = jnp.maximum((blk - mean) * inv * g_ref[...]
                                   + b_ref[...], 0.0)


def _tc_fused(self_feats, neigh_sum, W, gamma2, beta2):
    return pl.pallas_call(
        _tc_body,
        grid=(2 * NB,),
        in_specs=[
            pl.BlockSpec((BB, D), lambda i: (jnp.minimum(i, NB - 1), 0)),
            pl.BlockSpec((BB, D), lambda i: (jnp.minimum(i, NB - 1), 0)),
            pl.BlockSpec((D, 2 * D), lambda i: (0, 0)),
            pl.BlockSpec((D, 1), lambda i: (0, 0)),
            pl.BlockSpec((D, 1), lambda i: (0, 0)),
        ],
        out_specs=pl.BlockSpec((D, BB), lambda i: (0, jnp.maximum(i - NB, 0))),
        scratch_shapes=[
            pltpu.VMEM((D, B), jnp.float32),
            pltpu.VMEM((D, 1), jnp.float32),
            pltpu.VMEM((D, 1), jnp.float32),
        ],
        out_shape=jax.ShapeDtypeStruct((D, B), jnp.float32),
    )(self_feats, neigh_sum, W, gamma2, beta2)


def kernel(nodes, node_map, neigh_idx, features, W, gamma, beta):
    nidx = neigh_idx.reshape(NW, NCH, C * K)
    self_feats, neigh_sum = _make_sc_gather()(nodes, node_map, nidx, features)
    return _tc_fused(self_feats, neigh_sum, W,
                     gamma.reshape(D, 1), beta.reshape(D, 1))


# speedup vs baseline: 1.0141x; 1.0141x over previous
import functools

import jax
import jax.numpy as jnp
from jax import lax
from jax.experimental import pallas as pl
from jax.experimental.pallas import tpu as pltpu
from jax.experimental.pallas import tpu_sc as plsc

B = 8192
D = 256
K = 10
N = 50000
NC = 2
NS = 16
NW = NC * NS
BPW = B // NW
C = 8
NCH = BPW // C
SC = 64
NSC = BPW // SC


def _reduce_chunk(nbuf, sumbuf):
    def node_body(r, carry):
        rb = r * K
        for d in range(D // 16):
            acc = nbuf[rb, pl.ds(d * 16, 16)]
            for j in range(1, K):
                acc = acc + nbuf[rb + j, pl.ds(d * 16, 16)]
            sumbuf[r, pl.ds(d * 16, 16)] = acc
        return carry
    lax.fori_loop(0, C, node_body, 0)


RING = 2


def _sc_gather_body(bpw, nodes_h, nmap_h, nidx_h, feat_h, self_o, sum_o,
                    nid_v, map_v, nbr_v, sbuf0, sbuf1, nbuf0, nbuf1,
                    qbuf0, qbuf1, semi, sem_s0, sem_s1,
                    sem_n0, sem_n1, sem_o0, sem_o1):
    nch = bpw // C
    nsc = bpw // SC
    wid = lax.axis_index("s") * NC + lax.axis_index("c")
    base = wid * bpw

    cp_nbr = pltpu.async_copy(nidx_h.at[wid], nbr_v, semi)
    pltpu.sync_copy(nodes_h.at[pl.ds(base, bpw)], nid_v)
    pltpu.async_copy(nmap_h.at[nid_v], map_v, sem_s0).wait()
    cp_nbr.wait()

    nbufs = (nbuf0, nbuf1)
    nsems = (sem_n0, sem_n1)

    for b in range(RING):
        pltpu.async_copy(feat_h.at[nbr_v.at[b]], nbufs[b], nsems[b])

    sbufs, ssems = (sbuf0, sbuf1), (sem_s0, sem_s1)
    gathers = []
    for s in range(2):
        gathers.append(pltpu.async_copy(
            feat_h.at[map_v.at[pl.ds(s * SC, SC)]], sbufs[s], ssems[s]))
    outs = [None, None]
    for s in range(nsc):
        b = s % 2
        gathers[s].wait()
        if outs[b] is not None:
            outs[b].wait()
        outs[b] = pltpu.async_copy(
            sbufs[b], self_o.at[pl.ds(base + s * SC, SC)], ssems[b])
        if s + 2 < nsc:
            if outs[b] is not None:
                outs[b].wait()
                outs[b] = None
            gathers.append(pltpu.async_copy(
                feat_h.at[map_v.at[pl.ds((s + 2) * SC, SC)]],
                sbufs[b], ssems[b]))
    for o in outs:
        o.wait()

    qbufs = (qbuf0, qbuf1)
    osems = (sem_o0, sem_o1)

    def ring_body(p, carry):
        cc = p * RING
        for b in range(RING):
            c = cc + b
            q = b % 2
            pltpu.make_async_copy(feat_h.at[nbr_v.at[c]], nbufs[b],
                                  nsems[b]).wait()
            @pl.when(c >= 2)
            def _wait_prev():
                pltpu.make_async_copy(
                    qbufs[q], sum_o.at[pl.ds(base + (c - 2) * C, C)],
                    osems[q]).wait()
            _reduce_chunk(nbufs[b], qbufs[q])

            @pl.when(c + RING < nch)
            def _next_gather():
                pltpu.async_copy(feat_h.at[nbr_v.at[c + RING]], nbufs[b],
                                 nsems[b])
            pltpu.async_copy(qbufs[q], sum_o.at[pl.ds(base + c * C, C)],
                             osems[q])
        return carry

    lax.fori_loop(0, nch // RING, ring_body, 0)
    for b in range(2):
        pltpu.make_async_copy(
            qbufs[b], sum_o.at[pl.ds(base + (nch - 2 + b) * C, C)],
            osems[b]).wait()


@functools.cache
def _make_sc_gather(nb):
    return pl.kernel(
        functools.partial(_sc_gather_body, nb // NW),
        mesh=plsc.VectorSubcoreMesh(core_axis_name="c", subcore_axis_name="s"),
        out_type=[
            jax.ShapeDtypeStruct((nb, D), jnp.float32),
            jax.ShapeDtypeStruct((nb, D), jnp.float32),
        ],
        scratch_types=[
            pltpu.VMEM((nb // NW,), jnp.int32),
            pltpu.VMEM((nb // NW,), jnp.int32),
            pltpu.VMEM((nb // NW // C, C * K), jnp.int32),
            pltpu.VMEM((SC, D), jnp.float32),
            pltpu.VMEM((SC, D), jnp.float32),
            pltpu.VMEM((C * K, D), jnp.float32),
            pltpu.VMEM((C * K, D), jnp.float32),
            pltpu.VMEM((C, D), jnp.float32),
            pltpu.VMEM((C, D), jnp.float32),
            pltpu.SemaphoreType.DMA,
            pltpu.SemaphoreType.DMA,
            pltpu.SemaphoreType.DMA,
            pltpu.SemaphoreType.DMA,
            pltpu.SemaphoreType.DMA,
            pltpu.SemaphoreType.DMA,
            pltpu.SemaphoreType.DMA,
        ],
    )


NSPLIT = 2
B2 = B // NSPLIT
BB = 1024
NB2 = B2 // BB
NBO = B // BB


def _tc_mm_body(self_ref, sum_ref, w_ref, ht_ref, s1_ref, s2_ref):
    i = pl.program_id(0)
    ws = w_ref[:, :D]
    wn = w_ref[:, D:]
    dn = (((1,), (1,)), ((), ()))
    blk = lax.dot_general(ws, self_ref[...], dn,
                          preferred_element_type=jnp.float32)
    blk = blk + 0.1 * lax.dot_general(wn, sum_ref[...], dn,
                                      preferred_element_type=jnp.float32)
    ht_ref[...] = blk
    ps = jnp.sum(blk, axis=1, keepdims=True)
    pq = jnp.sum(blk * blk, axis=1, keepdims=True)

    @pl.when(i == 0)
    def _init():
        s1_ref[...] = ps
        s2_ref[...] = pq

    @pl.when(i > 0)
    def _acc():
        s1_ref[...] = s1_ref[...] + ps
        s2_ref[...] = s2_ref[...] + pq


def _tc_mm(self_feats, neigh_sum, W):
    return pl.pallas_call(
        _tc_mm_body,
        grid=(NB2,),
        in_specs=[
            pl.BlockSpec((BB, D), lambda i: (i, 0)),
            pl.BlockSpec((BB, D), lambda i: (i, 0)),
            pl.BlockSpec((D, 2 * D), lambda i: (0, 0)),
        ],
        out_specs=[
            pl.BlockSpec((D, BB), lambda i: (0, i)),
            pl.BlockSpec((D, 1), lambda i: (0, 0)),
            pl.BlockSpec((D, 1), lambda i: (0, 0)),
        ],
        out_shape=[
            jax.ShapeDtypeStruct((D, B2), jnp.float32),
            jax.ShapeDtypeStruct((D, 1), jnp.float32),
            jax.ShapeDtypeStruct((D, 1), jnp.float32),
        ],
    )(self_feats, neigh_sum, W)


def _tc_bn_body(ht0_ref, ht1_ref, s1a_ref, s1b_ref, s2a_ref, s2b_ref,
                g_ref, b_ref, out_ref):
    j = pl.program_id(0)
    mean = (s1a_ref[...] + s1b_ref[...]) * (1.0 / B)
    var = (s2a_ref[...] + s2b_ref[...]) * (1.0 / B) - mean * mean
    scale = lax.rsqrt(var + 1e-5) * g_ref[...]
    shift = b_ref[...] - mean * scale

    @pl.when(j < NB2)
    def _lo():
        out_ref[...] = jnp.maximum(ht0_ref[...] * scale + shift, 0.0)

    @pl.when(j >= NB2)
    def _hi():
        out_ref[...] = jnp.maximum(ht1_ref[...] * scale + shift, 0.0)


def _tc_bn(ht0, ht1, s1a, s1b, s2a, s2b, gamma2, beta2):
    return pl.pallas_call(
        _tc_bn_body,
        grid=(NBO,),
        in_specs=[
            pl.BlockSpec((D, BB), lambda j: (0, jnp.minimum(j, NB2 - 1))),
            pl.BlockSpec((D, BB), lambda j: (0, jnp.maximum(j - NB2, 0))),
            pl.BlockSpec((D, 1), lambda j: (0, 0)),
            pl.BlockSpec((D, 1), lambda j: (0, 0)),
            pl.BlockSpec((D, 1), lambda j: (0, 0)),
            pl.BlockSpec((D, 1), lambda j: (0, 0)),
            pl.BlockSpec((D, 1), lambda j: (0, 0)),
            pl.BlockSpec((D, 1), lambda j: (0, 0)),
        ],
        out_specs=pl.BlockSpec((D, BB), lambda j: (0, j)),
        out_shape=jax.ShapeDtypeStruct((D, B), jnp.float32),
    )(ht0, ht1, s1a, s1b, s2a, s2b, gamma2, beta2)


def kernel(nodes, node_map, neigh_idx, features, W, gamma, beta):
    sc = _make_sc_gather(B2)
    nidx = neigh_idx.reshape(NSPLIT, NW, B2 // NW // C, C * K)
    nodes2 = nodes.reshape(NSPLIT, B2)
    parts = []
    for s in range(NSPLIT):
        self_s, sum_s = sc(nodes2[s], node_map, nidx[s], features)
        parts.append(_tc_mm(self_s, sum_s, W))
    (ht0, s1a, s2a), (ht1, s1b, s2b) = parts
    return _tc_bn(ht0, ht1, s1a, s1b, s2a, s2b,
                  gamma.reshape(D, 1), beta.reshape(D, 1))


# speedup vs baseline: 1.0703x; 1.0554x over previous
import functools

import jax
import jax.numpy as jnp
from jax import lax
from jax.experimental import pallas as pl
from jax.experimental.pallas import tpu as pltpu
from jax.experimental.pallas import tpu_sc as plsc

B = 8192
D = 256
K = 10
N = 50000
NC = 2
NS = 16
NW = NC * NS
BPW = B // NW
C = 8
NCH = BPW // C
SC = 64
NSC = BPW // SC


def _reduce_chunk(nbuf, sumbuf):
    def node_body(r, carry):
        rb = r * K
        for d in range(D // 16):
            acc = nbuf[rb, pl.ds(d * 16, 16)]
            for j in range(1, K):
                acc = acc + nbuf[rb + j, pl.ds(d * 16, 16)]
            sumbuf[r, pl.ds(d * 16, 16)] = acc
        return carry
    lax.fori_loop(0, C, node_body, 0)


RING = 2


def _sc_gather_body(bpw, nodes_h, nmap_h, nidx_h, feat_h, self_o, sum_o,
                    nid_v, map_v, nbr_v, sbuf0, sbuf1, nbuf0, nbuf1,
                    qbuf0, qbuf1, semi, sem_s0, sem_s1,
                    sem_n0, sem_n1, sem_o0, sem_o1):
    nch = bpw // C
    nsc = bpw // SC
    wid = lax.axis_index("s") * NC + lax.axis_index("c")
    base = wid * bpw

    cp_nbr = pltpu.async_copy(nidx_h.at[wid], nbr_v, semi)
    pltpu.sync_copy(nodes_h.at[pl.ds(base, bpw)], nid_v)
    pltpu.async_copy(nmap_h.at[nid_v], map_v, sem_s0).wait()
    cp_nbr.wait()

    nbufs = (nbuf0, nbuf1)
    nsems = (sem_n0, sem_n1)

    for b in range(RING):
        pltpu.async_copy(feat_h.at[nbr_v.at[b]], nbufs[b], nsems[b])

    sbufs, ssems = (sbuf0, sbuf1), (sem_s0, sem_s1)
    gathers = []
    for s in range(2):
        gathers.append(pltpu.async_copy(
            feat_h.at[map_v.at[pl.ds(s * SC, SC)]], sbufs[s], ssems[s]))
    outs = [None, None]
    for s in range(nsc):
        b = s % 2
        gathers[s].wait()
        if outs[b] is not None:
            outs[b].wait()
        outs[b] = pltpu.async_copy(
            sbufs[b], self_o.at[pl.ds(base + s * SC, SC)], ssems[b])
        if s + 2 < nsc:
            if outs[b] is not None:
                outs[b].wait()
                outs[b] = None
            gathers.append(pltpu.async_copy(
                feat_h.at[map_v.at[pl.ds((s + 2) * SC, SC)]],
                sbufs[b], ssems[b]))
    for o in outs:
        o.wait()

    qbufs = (qbuf0, qbuf1)
    osems = (sem_o0, sem_o1)

    def ring_body(p, carry):
        cc = p * RING
        for b in range(RING):
            c = cc + b
            q = b % 2
            pltpu.make_async_copy(feat_h.at[nbr_v.at[c]], nbufs[b],
                                  nsems[b]).wait()
            @pl.when(c >= 2)
            def _wait_prev():
                pltpu.make_async_copy(
                    qbufs[q], sum_o.at[pl.ds(base + (c - 2) * C, C)],
                    osems[q]).wait()
            _reduce_chunk(nbufs[b], qbufs[q])

            @pl.when(c + RING < nch)
            def _next_gather():
                pltpu.async_copy(feat_h.at[nbr_v.at[c + RING]], nbufs[b],
                                 nsems[b])
            pltpu.async_copy(qbufs[q], sum_o.at[pl.ds(base + c * C, C)],
                             osems[q])
        return carry

    lax.fori_loop(0, nch // RING, ring_body, 0)
    for b in range(2):
        pltpu.make_async_copy(
            qbufs[b], sum_o.at[pl.ds(base + (nch - 2 + b) * C, C)],
            osems[b]).wait()


@functools.cache
def _make_sc_gather(nb):
    return pl.kernel(
        functools.partial(_sc_gather_body, nb // NW),
        mesh=plsc.VectorSubcoreMesh(core_axis_name="c", subcore_axis_name="s"),
        out_type=[
            jax.ShapeDtypeStruct((nb, D), jnp.float32),
            jax.ShapeDtypeStruct((nb, D), jnp.float32),
        ],
        scratch_types=[
            pltpu.VMEM((nb // NW,), jnp.int32),
            pltpu.VMEM((nb // NW,), jnp.int32),
            pltpu.VMEM((nb // NW // C, C * K), jnp.int32),
            pltpu.VMEM((SC, D), jnp.float32),
            pltpu.VMEM((SC, D), jnp.float32),
            pltpu.VMEM((C * K, D), jnp.float32),
            pltpu.VMEM((C * K, D), jnp.float32),
            pltpu.VMEM((C, D), jnp.float32),
            pltpu.VMEM((C, D), jnp.float32),
            pltpu.SemaphoreType.DMA,
            pltpu.SemaphoreType.DMA,
            pltpu.SemaphoreType.DMA,
            pltpu.SemaphoreType.DMA,
            pltpu.SemaphoreType.DMA,
            pltpu.SemaphoreType.DMA,
            pltpu.SemaphoreType.DMA,
        ],
    )


NB = 16
BB = B // NB


def _tc_body(self_ref, sum_ref, w_ref, g_ref, b_ref, out_ref,
             ht_ref, s1_ref, s2_ref):
    i = pl.program_id(0)

    @pl.when(i < NB)
    def _matmul():
        ws = w_ref[:, :D]
        wn = w_ref[:, D:]
        dn = (((1,), (1,)), ((), ()))
        blk = lax.dot_general(ws, self_ref[...], dn,
                              preferred_element_type=jnp.float32)
        blk = blk + 0.1 * lax.dot_general(wn, sum_ref[...], dn,
                                          preferred_element_type=jnp.float32)
        ht_ref[:, pl.ds(i * BB, BB)] = blk
        ps = jnp.sum(blk, axis=1, keepdims=True)
        pq = jnp.sum(blk * blk, axis=1, keepdims=True)

        @pl.when(i == 0)
        def _init():
            s1_ref[...] = ps
            s2_ref[...] = pq

        @pl.when(i > 0)
        def _acc():
            s1_ref[...] = s1_ref[...] + ps
            s2_ref[...] = s2_ref[...] + pq

    @pl.when(i >= NB)
    def _bn():
        j = i - NB
        mean = s1_ref[...] * (1.0 / B)
        var = s2_ref[...] * (1.0 / B) - mean * mean
        inv = lax.rsqrt(var + 1e-5)
        blk = ht_ref[:, pl.ds(j * BB, BB)]
        out_ref[...] = jnp.maximum((blk - mean) * inv * g_ref[...]
                                   + b_ref[...], 0.0)


def _tc_fused(self_feats, neigh_sum, W, gamma2, beta2):
    return pl.pallas_call(
        _tc_body,
        grid=(2 * NB,),
        in_specs=[
            pl.BlockSpec((BB, D), lambda i: (jnp.minimum(i, NB - 1), 0)),
            pl.BlockSpec((BB, D), lambda i: (jnp.minimum(i, NB - 1), 0)),
            pl.BlockSpec((D, 2 * D), lambda i: (0, 0)),
            pl.BlockSpec((D, 1), lambda i: (0, 0)),
            pl.BlockSpec((D, 1), lambda i: (0, 0)),
        ],
        out_specs=pl.BlockSpec((D, BB), lambda i: (0, jnp.maximum(i - NB, 0))),
        scratch_shapes=[
            pltpu.VMEM((D, B), jnp.float32),
            pltpu.VMEM((D, 1), jnp.float32),
            pltpu.VMEM((D, 1), jnp.float32),
        ],
        out_shape=jax.ShapeDtypeStruct((D, B), jnp.float32),
    )(self_feats, neigh_sum, W, gamma2, beta2)


def kernel(nodes, node_map, neigh_idx, features, W, gamma, beta):
    nidx = neigh_idx.reshape(NW, B // NW // C, C * K)
    self_feats, neigh_sum = _make_sc_gather(B)(nodes, node_map, nidx,
                                               features)
    return _tc_fused(self_feats, neigh_sum, W,
                     gamma.reshape(D, 1), beta.reshape(D, 1))


# speedup vs baseline: 1.1777x; 1.1003x over previous
import functools

import jax
import jax.numpy as jnp
from jax import lax
from jax.experimental import pallas as pl
from jax.experimental.pallas import tpu as pltpu
from jax.experimental.pallas import tpu_sc as plsc

B = 8192
D = 256
K = 10
N = 50000
NC = 2
NS = 16
NW = NC * NS
BPW = B // NW
C = 8
NCH = BPW // C
SC = 64
NSC = BPW // SC


def _reduce_chunk(nbuf, sumbuf):
    def node_body(r, carry):
        rb = r * K
        for d in range(D // 16):
            acc = nbuf[rb, pl.ds(d * 16, 16)]
            for j in range(1, K):
                acc = acc + nbuf[rb + j, pl.ds(d * 16, 16)]
            sumbuf[r, pl.ds(d * 16, 16)] = acc
        return carry
    lax.fori_loop(0, C, node_body, 0)


RING = 2


def _sc_gather_body(bpw, nodes_h, nmap_h, nidx_h, feat_h, self_o, sum_o,
                    nid_v, map_v, nbr_v, sbuf0, sbuf1, nbuf0, nbuf1,
                    qbuf0, qbuf1, semi, sem_s0, sem_s1,
                    sem_n0, sem_n1, sem_o0, sem_o1):
    nch = bpw // C
    nsc = bpw // SC
    wid = lax.axis_index("s") * NC + lax.axis_index("c")
    base = wid * bpw

    cp_nbr = pltpu.async_copy(nidx_h.at[wid], nbr_v, semi)
    pltpu.sync_copy(nodes_h.at[pl.ds(base, bpw)], nid_v)
    pltpu.async_copy(nmap_h.at[nid_v], map_v, sem_s0).wait()
    cp_nbr.wait()

    nbufs = (nbuf0, nbuf1)
    nsems = (sem_n0, sem_n1)

    for b in range(RING):
        pltpu.async_copy(feat_h.at[nbr_v.at[b]], nbufs[b], nsems[b])

    sbufs, ssems = (sbuf0, sbuf1), (sem_s0, sem_s1)
    gathers = []
    for s in range(2):
        gathers.append(pltpu.async_copy(
            feat_h.at[map_v.at[pl.ds(s * SC, SC)]], sbufs[s], ssems[s]))
    outs = [None, None]
    for s in range(nsc):
        b = s % 2
        gathers[s].wait()
        if outs[b] is not None:
            outs[b].wait()
        outs[b] = pltpu.async_copy(
            sbufs[b], self_o.at[pl.ds(base + s * SC, SC)], ssems[b])
        if s + 2 < nsc:
            if outs[b] is not None:
                outs[b].wait()
                outs[b] = None
            gathers.append(pltpu.async_copy(
                feat_h.at[map_v.at[pl.ds((s + 2) * SC, SC)]],
                sbufs[b], ssems[b]))
    for o in outs:
        o.wait()

    qbufs = (qbuf0, qbuf1)
    osems = (sem_o0, sem_o1)

    def ring_body(p, carry):
        cc = p * RING
        for b in range(RING):
            c = cc + b
            q = b % 2
            pltpu.make_async_copy(feat_h.at[nbr_v.at[c]], nbufs[b],
                                  nsems[b]).wait()
            @pl.when(c >= 2)
            def _wait_prev():
                pltpu.make_async_copy(
                    qbufs[q], sum_o.at[pl.ds(base + (c - 2) * C, C)],
                    osems[q]).wait()
            _reduce_chunk(nbufs[b], qbufs[q])

            @pl.when(c + RING < nch)
            def _next_gather():
                pltpu.async_copy(feat_h.at[nbr_v.at[c + RING]], nbufs[b],
                                 nsems[b])
            pltpu.async_copy(qbufs[q], sum_o.at[pl.ds(base + c * C, C)],
                             osems[q])
        return carry

    lax.fori_loop(0, nch // RING, ring_body, 0)
    for b in range(2):
        pltpu.make_async_copy(
            qbufs[b], sum_o.at[pl.ds(base + (nch - 2 + b) * C, C)],
            osems[b]).wait()


@functools.cache
def _make_sc_gather(nb):
    return pl.kernel(
        functools.partial(_sc_gather_body, nb // NW),
        mesh=plsc.VectorSubcoreMesh(core_axis_name="c", subcore_axis_name="s"),
        out_type=[
            jax.ShapeDtypeStruct((nb, D), jnp.float32),
            jax.ShapeDtypeStruct((nb, D), jnp.float32),
        ],
        scratch_types=[
            pltpu.VMEM((nb // NW,), jnp.int32),
            pltpu.VMEM((nb // NW,), jnp.int32),
            pltpu.VMEM((nb // NW // C, C * K), jnp.int32),
            pltpu.VMEM((SC, D), jnp.float32),
            pltpu.VMEM((SC, D), jnp.float32),
            pltpu.VMEM((C * K, D), jnp.float32),
            pltpu.VMEM((C * K, D), jnp.float32),
            pltpu.VMEM((C, D), jnp.float32),
            pltpu.VMEM((C, D), jnp.float32),
            pltpu.SemaphoreType.DMA,
            pltpu.SemaphoreType.DMA,
            pltpu.SemaphoreType.DMA,
            pltpu.SemaphoreType.DMA,
            pltpu.SemaphoreType.DMA,
            pltpu.SemaphoreType.DMA,
            pltpu.SemaphoreType.DMA,
        ],
    )


NB = 4
BB = B // NB


def _tc_body(self_ref, sum_ref, w_ref, g_ref, b_ref, out_ref,
             ht_ref, s1_ref, s2_ref):
    i = pl.program_id(0)

    @pl.when(i < NB)
    def _matmul():
        ws = w_ref[:, :D]
        wn = w_ref[:, D:]
        dn = (((1,), (1,)), ((), ()))
        blk = lax.dot_general(ws, self_ref[...], dn,
                              preferred_element_type=jnp.float32)
        blk = blk + 0.1 * lax.dot_general(wn, sum_ref[...], dn,
                                          preferred_element_type=jnp.float32)
        ht_ref[:, pl.ds(i * BB, BB)] = blk
        ps = jnp.sum(blk, axis=1, keepdims=True)
        pq = jnp.sum(blk * blk, axis=1, keepdims=True)

        @pl.when(i == 0)
        def _init():
            s1_ref[...] = ps
            s2_ref[...] = pq

        @pl.when(i > 0)
        def _acc():
            s1_ref[...] = s1_ref[...] + ps
            s2_ref[...] = s2_ref[...] + pq

    @pl.when(i >= NB)
    def _bn():
        j = i - NB
        mean = s1_ref[...] * (1.0 / B)
        var = s2_ref[...] * (1.0 / B) - mean * mean
        inv = lax.rsqrt(var + 1e-5)
        blk = ht_ref[:, pl.ds(j * BB, BB)]
        out_ref[...] = jnp.maximum((blk - mean) * inv * g_ref[...]
                                   + b_ref[...], 0.0)


def _tc_fused(self_feats, neigh_sum, W, gamma2, beta2):
    return pl.pallas_call(
        _tc_body,
        grid=(2 * NB,),
        in_specs=[
            pl.BlockSpec((BB, D), lambda i: (jnp.minimum(i, NB - 1), 0)),
            pl.BlockSpec((BB, D), lambda i: (jnp.minimum(i, NB - 1), 0)),
            pl.BlockSpec((D, 2 * D), lambda i: (0, 0)),
            pl.BlockSpec((D, 1), lambda i: (0, 0)),
            pl.BlockSpec((D, 1), lambda i: (0, 0)),
        ],
        out_specs=pl.BlockSpec((D, BB), lambda i: (0, jnp.maximum(i - NB, 0))),
        scratch_shapes=[
            pltpu.VMEM((D, B), jnp.float32),
            pltpu.VMEM((D, 1), jnp.float32),
            pltpu.VMEM((D, 1), jnp.float32),
        ],
        out_shape=jax.ShapeDtypeStruct((D, B), jnp.float32),
    )(self_feats, neigh_sum, W, gamma2, beta2)


def kernel(nodes, node_map, neigh_idx, features, W, gamma, beta):
    nidx = neigh_idx.reshape(NW, B // NW // C, C * K)
    self_feats, neigh_sum = _make_sc_gather(B)(nodes, node_map, nidx,
                                               features)
    return _tc_fused(self_feats, neigh_sum, W,
                     gamma.reshape(D, 1), beta.reshape(D, 1))


# speedup vs baseline: 1.1901x; 1.0106x over previous
import functools

import jax
import jax.numpy as jnp
from jax import lax
from jax.experimental import pallas as pl
from jax.experimental.pallas import tpu as pltpu
from jax.experimental.pallas import tpu_sc as plsc

B = 8192
D = 256
K = 10
N = 50000
NC = 2
NS = 16
NW = NC * NS
BPW = B // NW
C = 8
NCH = BPW // C
SC = 64
NSC = BPW // SC


def _reduce_chunk(nbuf, sumbuf):
    def node_body(r, carry):
        rb = r * K
        for d in range(D // 16):
            acc = nbuf[rb, pl.ds(d * 16, 16)]
            for j in range(1, K):
                acc = acc + nbuf[rb + j, pl.ds(d * 16, 16)]
            sumbuf[r, pl.ds(d * 16, 16)] = acc
        return carry
    lax.fori_loop(0, C, node_body, 0)


RING = 2


def _sc_gather_body(bpw, nodes_h, nmap_h, nidx_h, feat_h, self_o, sum_o,
                    nid_v, map_v, nbr_v, sbuf0, sbuf1, nbuf0, nbuf1,
                    qbuf0, qbuf1, semi, sem_s0, sem_s1,
                    sem_n0, sem_n1, sem_o0, sem_o1):
    nch = bpw // C
    nsc = bpw // SC
    wid = lax.axis_index("s") * NC + lax.axis_index("c")
    base = wid * bpw

    cp_nbr = pltpu.async_copy(nidx_h.at[wid], nbr_v, semi)
    pltpu.sync_copy(nodes_h.at[pl.ds(base, bpw)], nid_v)
    pltpu.async_copy(nmap_h.at[nid_v], map_v, sem_s0).wait()
    cp_nbr.wait()

    nbufs = (nbuf0, nbuf1)
    nsems = (sem_n0, sem_n1)

    for b in range(RING):
        pltpu.async_copy(feat_h.at[nbr_v.at[b]], nbufs[b], nsems[b])

    sbufs, ssems = (sbuf0, sbuf1), (sem_s0, sem_s1)
    gathers = []
    for s in range(2):
        gathers.append(pltpu.async_copy(
            feat_h.at[map_v.at[pl.ds(s * SC, SC)]], sbufs[s], ssems[s]))
    outs = [None, None]
    for s in range(nsc):
        b = s % 2
        gathers[s].wait()
        if outs[b] is not None:
            outs[b].wait()
        outs[b] = pltpu.async_copy(
            sbufs[b], self_o.at[pl.ds(base + s * SC, SC)], ssems[b])
        if s + 2 < nsc:
            if outs[b] is not None:
                outs[b].wait()
                outs[b] = None
            gathers.append(pltpu.async_copy(
                feat_h.at[map_v.at[pl.ds((s + 2) * SC, SC)]],
                sbufs[b], ssems[b]))
    for o in outs:
        o.wait()

    qbufs = (qbuf0, qbuf1)
    osems = (sem_o0, sem_o1)

    def ring_body(p, carry):
        cc = p * RING
        for b in range(RING):
            c = cc + b
            q = b % 2
            pltpu.make_async_copy(feat_h.at[nbr_v.at[c]], nbufs[b],
                                  nsems[b]).wait()
            @pl.when(c >= 2)
            def _wait_prev():
                pltpu.make_async_copy(
                    qbufs[q], sum_o.at[pl.ds(base + (c - 2) * C, C)],
                    osems[q]).wait()
            _reduce_chunk(nbufs[b], qbufs[q])

            @pl.when(c + RING < nch)
            def _next_gather():
                pltpu.async_copy(feat_h.at[nbr_v.at[c + RING]], nbufs[b],
                                 nsems[b])
            pltpu.async_copy(qbufs[q], sum_o.at[pl.ds(base + c * C, C)],
                             osems[q])
        return carry

    lax.fori_loop(0, nch // RING, ring_body, 0)
    for b in range(2):
        pltpu.make_async_copy(
            qbufs[b], sum_o.at[pl.ds(base + (nch - 2 + b) * C, C)],
            osems[b]).wait()


@functools.cache
def _make_sc_gather(nb):
    return pl.kernel(
        functools.partial(_sc_gather_body, nb // NW),
        mesh=plsc.VectorSubcoreMesh(core_axis_name="c", subcore_axis_name="s"),
        out_type=[
            jax.ShapeDtypeStruct((nb, D), jnp.float32),
            jax.ShapeDtypeStruct((nb, D), jnp.float32),
        ],
        scratch_types=[
            pltpu.VMEM((nb // NW,), jnp.int32),
            pltpu.VMEM((nb // NW,), jnp.int32),
            pltpu.VMEM((nb // NW // C, C * K), jnp.int32),
            pltpu.VMEM((SC, D), jnp.float32),
            pltpu.VMEM((SC, D), jnp.float32),
            pltpu.VMEM((C * K, D), jnp.float32),
            pltpu.VMEM((C * K, D), jnp.float32),
            pltpu.VMEM((C, D), jnp.float32),
            pltpu.VMEM((C, D), jnp.float32),
            pltpu.SemaphoreType.DMA,
            pltpu.SemaphoreType.DMA,
            pltpu.SemaphoreType.DMA,
            pltpu.SemaphoreType.DMA,
            pltpu.SemaphoreType.DMA,
            pltpu.SemaphoreType.DMA,
            pltpu.SemaphoreType.DMA,
        ],
    )


NB = 2
BB = B // NB


def _tc_body(self_ref, sum_ref, w_ref, g_ref, b_ref, out_ref,
             ht_ref, s1_ref, s2_ref):
    i = pl.program_id(0)

    @pl.when(i < NB)
    def _matmul():
        ws = w_ref[:, :D]
        wn = w_ref[:, D:]
        dn = (((1,), (1,)), ((), ()))
        blk = lax.dot_general(ws, self_ref[...], dn,
                              preferred_element_type=jnp.float32)
        blk = blk + 0.1 * lax.dot_general(wn, sum_ref[...], dn,
                                          preferred_element_type=jnp.float32)
        ht_ref[:, pl.ds(i * BB, BB)] = blk
        ps = jnp.sum(blk, axis=1, keepdims=True)
        pq = jnp.sum(blk * blk, axis=1, keepdims=True)

        @pl.when(i == 0)
        def _init():
            s1_ref[...] = ps
            s2_ref[...] = pq

        @pl.when(i > 0)
        def _acc():
            s1_ref[...] = s1_ref[...] + ps
            s2_ref[...] = s2_ref[...] + pq

    @pl.when(i >= NB)
    def _bn():
        j = i - NB
        mean = s1_ref[...] * (1.0 / B)
        var = s2_ref[...] * (1.0 / B) - mean * mean
        inv = lax.rsqrt(var + 1e-5)
        blk = ht_ref[:, pl.ds(j * BB, BB)]
        out_ref[...] = jnp.maximum((blk - mean) * inv * g_ref[...]
                                   + b_ref[...], 0.0)


def _tc_fused(self_feats, neigh_sum, W, gamma2, beta2):
    return pl.pallas_call(
        _tc_body,
        grid=(2 * NB,),
        in_specs=[
            pl.BlockSpec((BB, D), lambda i: (jnp.minimum(i, NB - 1), 0)),
            pl.BlockSpec((BB, D), lambda i: (jnp.minimum(i, NB - 1), 0)),
            pl.BlockSpec((D, 2 * D), lambda i: (0, 0)),
            pl.BlockSpec((D, 1), lambda i: (0, 0)),
            pl.BlockSpec((D, 1), lambda i: (0, 0)),
        ],
        out_specs=pl.BlockSpec((D, BB), lambda i: (0, jnp.maximum(i - NB, 0))),
        scratch_shapes=[
            pltpu.VMEM((D, B), jnp.float32),
            pltpu.VMEM((D, 1), jnp.float32),
            pltpu.VMEM((D, 1), jnp.float32),
        ],
        out_shape=jax.ShapeDtypeStruct((D, B), jnp.float32),
    )(self_feats, neigh_sum, W, gamma2, beta2)


def kernel(nodes, node_map, neigh_idx, features, W, gamma, beta):
    nidx = neigh_idx.reshape(NW, B // NW // C, C * K)
    self_feats, neigh_sum = _make_sc_gather(B)(nodes, node_map, nidx,
                                               features)
    return _tc_fused(self_feats, neigh_sum, W,
                     gamma.reshape(D, 1), beta.reshape(D, 1))
